# Initial kernel scaffold; baseline (speedup 1.0000x reference)
import jax
import jax.numpy as jnp
from jax.experimental import pallas as pl


def kernel(predicate_indices_list, motif_counts_batch, table, W1, b1, W2, b2, W3, b3):
    raise NotImplementedError("write your pallas kernel here")


# speedup vs baseline: 2.3974x; 2.3974x over previous
import functools

import jax
import jax.numpy as jnp
from jax import lax
from jax.experimental import pallas as pl
from jax.experimental.pallas import tpu as pltpu
from jax.experimental.pallas import tpu_sc as plsc

_B, _L, _V, _D = 16384, 50, 1000000, 64
_M, _MD, _O = 100, 64, 128

_NC, _NS = 2, 16
_NW = _NC * _NS
_RPW = _B // _NW
_RC = 16
_IPD = 100
_DPC = _RC * _L // _IPD
_NCHUNK = _RPW // _RC
_IDX_ROWS_PER_CHUNK = _RC * _L // _IPD


def _gather_max_body(idx_hbm, table_hbm, out_hbm, idx_v, rows_v, out_v, sem):
    wid = lax.axis_index("s") * _NC + lax.axis_index("c")
    rule0 = wid * _RPW
    idxrow0 = wid * (_RPW * _L // _IPD)

    def chunk_body(g, carry):
        rb = rule0 + g * _RC
        ib = idxrow0 + g * _IDX_ROWS_PER_CHUNK
        pltpu.sync_copy(idx_hbm.at[pl.ds(ib, _IDX_ROWS_PER_CHUNK)], idx_v)
        copies = []
        for j in range(_DPC):
            copies.append(
                pltpu.async_copy(
                    table_hbm.at[idx_v.at[j]],
                    rows_v.at[pl.ds(j * _IPD, _IPD)],
                    sem,
                )
            )
        for c in copies:
            c.wait()

        def rule_body(r, carry2):
            base = r * _L
            accs = [rows_v[base, pl.ds(cg * 16, 16)] for cg in range(_D // 16)]
            for l in range(1, _L):
                for cg in range(_D // 16):
                    accs[cg] = jnp.maximum(
                        accs[cg], rows_v[base + l, pl.ds(cg * 16, 16)]
                    )
            for cg in range(_D // 16):
                out_v[r, pl.ds(cg * 16, 16)] = accs[cg]
            return carry2

        lax.fori_loop(0, _RC, rule_body, 0, unroll=False)
        pltpu.sync_copy(out_v, out_hbm.at[pl.ds(rb, _RC)])
        return carry

    lax.fori_loop(0, _NCHUNK, chunk_body, 0, unroll=False)


@jax.jit
def _gather_max(idx2d, table):
    mesh = plsc.VectorSubcoreMesh(core_axis_name="c", subcore_axis_name="s")
    return pl.kernel(
        _gather_max_body,
        out_type=jax.ShapeDtypeStruct((_B, _D), jnp.float32),
        mesh=mesh,
        scratch_types=[
            pltpu.VMEM((_IDX_ROWS_PER_CHUNK, _IPD), jnp.int32),
            pltpu.VMEM((_RC * _L, _D), jnp.float32),
            pltpu.VMEM((_RC, _D), jnp.float32),
            pltpu.SemaphoreType.DMA,
        ],
        compiler_params=pltpu.CompilerParams(use_tc_tiling_on_sc=False),
    )(idx2d, table)


def _mlp_body(pred_ref, motif_ref, w1_ref, b1_ref, w2_ref, b2_ref,
              w3a_ref, w3b_ref, b3_ref, out_ref):
    h = jnp.dot(motif_ref[...], w1_ref[...], preferred_element_type=jnp.float32)
    h = jnp.maximum(h + b1_ref[...], 0.0)
    m = jnp.dot(h, w2_ref[...], preferred_element_type=jnp.float32)
    m = jnp.maximum(m + b2_ref[...], 0.0)
    o = jnp.dot(pred_ref[...], w3a_ref[...], preferred_element_type=jnp.float32)
    o = o + jnp.dot(m, w3b_ref[...], preferred_element_type=jnp.float32)
    out_ref[...] = jnp.maximum(o + b3_ref[...], 0.0)


@jax.jit
def _mlp(pred, motif, W1, b1, W2, b2, W3a, W3b, b3):
    bb = 2048
    grid = (_B // bb,)
    rep = lambda shape: pl.BlockSpec(shape, lambda i: (0,) * len(shape))
    return pl.pallas_call(
        _mlp_body,
        grid=grid,
        in_specs=[
            pl.BlockSpec((bb, _D), lambda i: (i, 0)),
            pl.BlockSpec((bb, _M), lambda i: (i, 0)),
            rep((_M, _MD)), rep((1, _MD)),
            rep((_MD, _MD)), rep((1, _MD)),
            rep((_D, _O)), rep((_MD, _O)), rep((1, _O)),
        ],
        out_specs=pl.BlockSpec((bb, _O), lambda i: (i, 0)),
        out_shape=jax.ShapeDtypeStruct((_B, _O), jnp.float32),
    )(pred, motif, W1, b1, W2, b2, W3a, W3b, b3)


def kernel(predicate_indices_list, motif_counts_batch, table, W1, b1, W2, b2, W3, b3):
    idx2d = predicate_indices_list.reshape(_B * _L // _IPD, _IPD)
    pred = _gather_max(idx2d, table)
    return _mlp(
        pred, motif_counts_batch,
        W1, b1.reshape(1, _MD),
        W2, b2.reshape(1, _MD),
        W3[:_D], W3[_D:], b3.reshape(1, _O),
    )


# speedup vs baseline: 2.7105x; 1.1306x over previous
import functools

import jax
import jax.numpy as jnp
from jax import lax
from jax.experimental import pallas as pl
from jax.experimental.pallas import tpu as pltpu
from jax.experimental.pallas import tpu_sc as plsc

_B, _L, _V, _D = 16384, 50, 1000000, 64
_M, _MD, _O = 100, 64, 128

_NC, _NS = 2, 16
_NW = _NC * _NS
_RPW = _B // _NW
_RC = 8
_IPD = 80
_CIDX = _RC * _L
_DPC = _CIDX // _IPD
_NCHUNK = _RPW // _RC


def _gather_max_body(idx_hbm, table_hbm, out_hbm,
                     idx_v, rows0, rows1, out_v, sem0, sem1):
    wid = lax.axis_index("s") * _NC + lax.axis_index("c")
    rule0 = wid * _RPW

    pltpu.sync_copy(idx_hbm.at[pl.ds(rule0 * _L, _RPW * _L)], idx_v)

    def fire(c, rows, sem):
        for j in range(_DPC):
            pltpu.async_copy(
                table_hbm.at[idx_v.at[pl.ds(c * _CIDX + j * _IPD, _IPD)]],
                rows.at[pl.ds(j * _IPD, _IPD)],
                sem,
            )

    def compute(c, rows):
        def rule_body(r, carry):
            base = r * _L
            accs = [rows[base, pl.ds(cg * 16, 16)] for cg in range(_D // 16)]
            for l in range(1, _L):
                for cg in range(_D // 16):
                    accs[cg] = jnp.maximum(
                        accs[cg], rows[base + l, pl.ds(cg * 16, 16)]
                    )
            for cg in range(_D // 16):
                out_v[c * _RC + r, pl.ds(cg * 16, 16)] = accs[cg]
            return carry

        lax.fori_loop(0, _RC, rule_body, 0, unroll=False)

    def half(c, rows_cur, sem_cur, rows_nxt, sem_nxt):
        @pl.when(c + 1 < _NCHUNK)
        def _():
            fire(c + 1, rows_nxt, sem_nxt)

        pltpu.make_async_copy(
            table_hbm.at[pl.ds(0, _CIDX)], rows_cur, sem_cur
        ).wait()
        compute(c, rows_cur)

    fire(0, rows0, sem0)

    def pair(k, carry):
        half(2 * k, rows0, sem0, rows1, sem1)
        half(2 * k + 1, rows1, sem1, rows0, sem0)
        return carry

    lax.fori_loop(0, _NCHUNK // 2, pair, 0, unroll=False)

    pltpu.sync_copy(out_v, out_hbm.at[pl.ds(rule0, _RPW)])


@jax.jit
def _gather_max(idx_flat, table):
    mesh = plsc.VectorSubcoreMesh(core_axis_name="c", subcore_axis_name="s")
    return pl.kernel(
        _gather_max_body,
        out_type=jax.ShapeDtypeStruct((_B, _D), jnp.float32),
        mesh=mesh,
        scratch_types=[
            pltpu.VMEM((_RPW * _L,), jnp.int32),
            pltpu.VMEM((_CIDX, _D), jnp.float32),
            pltpu.VMEM((_CIDX, _D), jnp.float32),
            pltpu.VMEM((_RPW, _D), jnp.float32),
            pltpu.SemaphoreType.DMA,
            pltpu.SemaphoreType.DMA,
        ],
        compiler_params=pltpu.CompilerParams(use_tc_tiling_on_sc=False),
    )(idx_flat, table)


def _mlp_body(pred_ref, motif_ref, w1_ref, b1_ref, w2_ref, b2_ref,
              w3a_ref, w3b_ref, b3_ref, out_ref):
    h = jnp.dot(motif_ref[...], w1_ref[...], preferred_element_type=jnp.float32)
    h = jnp.maximum(h + b1_ref[...], 0.0)
    m = jnp.dot(h, w2_ref[...], preferred_element_type=jnp.float32)
    m = jnp.maximum(m + b2_ref[...], 0.0)
    o = jnp.dot(pred_ref[...], w3a_ref[...], preferred_element_type=jnp.float32)
    o = o + jnp.dot(m, w3b_ref[...], preferred_element_type=jnp.float32)
    out_ref[...] = jnp.maximum(o + b3_ref[...], 0.0)


@jax.jit
def _mlp(pred, motif, W1, b1, W2, b2, W3a, W3b, b3):
    bb = 2048
    grid = (_B // bb,)
    rep = lambda shape: pl.BlockSpec(shape, lambda i: (0,) * len(shape))
    return pl.pallas_call(
        _mlp_body,
        grid=grid,
        in_specs=[
            pl.BlockSpec((bb, _D), lambda i: (i, 0)),
            pl.BlockSpec((bb, _M), lambda i: (i, 0)),
            rep((_M, _MD)), rep((1, _MD)),
            rep((_MD, _MD)), rep((1, _MD)),
            rep((_D, _O)), rep((_MD, _O)), rep((1, _O)),
        ],
        out_specs=pl.BlockSpec((bb, _O), lambda i: (i, 0)),
        out_shape=jax.ShapeDtypeStruct((_B, _O), jnp.float32),
    )(pred, motif, W1, b1, W2, b2, W3a, W3b, b3)


def kernel(predicate_indices_list, motif_counts_batch, table, W1, b1, W2, b2, W3, b3):
    idx_flat = predicate_indices_list.reshape(_B * _L)
    pred = _gather_max(idx_flat, table)
    return _mlp(
        pred, motif_counts_batch,
        W1, b1.reshape(1, _MD),
        W2, b2.reshape(1, _MD),
        W3[:_D], W3[_D:], b3.reshape(1, _O),
    )
